# Initial kernel scaffold; baseline (speedup 1.0000x reference)
import jax
import jax.numpy as jnp
from jax.experimental import pallas as pl


def kernel(x, y, W_hq, W1k, W2k, W1v, W2v, W_dot, in_proj_w, in_proj_b, out_proj_w, out_proj_b):
    raise NotImplementedError("write your pallas kernel here")


# speedup vs baseline: 75.1655x; 75.1655x over previous
import functools
import math

import jax
import jax.numpy as jnp
import numpy as np
from jax.experimental import pallas as pl
from jax.experimental.pallas import tpu as pltpu

RADIUS = 2.0
NB = 10
NHEAD = 8
HI = jax.lax.Precision.HIGHEST

_C = 64
_N = 1024
_SOH_STEP = RADIUS / (NB + 1)
_SOH_CONST = 1.14136 * float(np.exp(2.0))


def _sus(x):
    return jnp.where(x > 0.0, jnp.exp(-1.0 / jnp.where(x > 0.0, x, 1.0)), 0.0)


def _prep_body(f_ref, ft_ref, wkrt_ref, wv_ref, whq_ref, wdot_ref,
               akt_ref, vtab_ref, qws_ref, *, s_q):
    f = f_ref[0]
    ft = ft_ref[0]
    akt_ref[0] = jnp.dot(wkrt_ref[...], ft, precision=HI)
    for c in range(16):
        vtab_ref[0, c] = jnp.dot(f, wv_ref[c], precision=HI)
    wqd = jnp.dot(whq_ref[...], wdot_ref[...], precision=HI)
    qws_ref[0] = jnp.dot(f, wqd, precision=HI) * s_q


def _edge_body(post_ref, posd_ref, qws_ref, akt_ref, vtab_ref,
               w1k_ref, w1v_ref, o0_ref, o1_ref):
    j = pl.program_id(1)
    post = post_ref[0]
    posd = posd_ref[0]
    qws = qws_ref[0]
    akt = akt_ref[0]
    vtab = vtab_ref[0]

    p2 = jnp.dot(qws, akt, precision=HI)

    dx = post[0:1, :] - posd[:, 0:1]
    dy = post[1:2, :] - posd[:, 1:2]
    dz = post[2:3, :] - posd[:, 2:3]
    len2 = dx * dx + dy * dy + dz * dz
    elen = jnp.sqrt(len2)
    inv = jax.lax.rsqrt(jnp.where(len2 > 0.0, len2, 1.0))
    ux = dx * inv
    uy = dy * inv
    uz = dz * inv

    src = jax.lax.broadcasted_iota(jnp.int32, (_C, _N), 1)
    dst = jax.lax.broadcasted_iota(jnp.int32, (_C, _N), 0) + j * _C
    mask = (elen < RADIUS) & (src != dst)
    cutoff = _sus(10.0 - 5.0 * elen)

    soh = []
    for b in range(NB):
        diff = (elen - (b + 1) * _SOH_STEP) * (1.0 / _SOH_STEP)
        soh.append(_SOH_CONST * _sus(diff + 1.0) * _sus(1.0 - diff))

    dotqk = jnp.zeros((_C, _N), jnp.float32)
    for c in range(16):
        acc = soh[0] * w1k_ref[0, c]
        for b in range(1, NB):
            acc = acc + soh[b] * w1k_ref[b, c]
        hk = acc * (1.0 / (1.0 + jnp.exp(-acc)))
        dotqk = dotqk + hk * p2[:, c * _N:(c + 1) * _N]

    ex = jnp.where(mask, cutoff * jnp.exp(dotqk), 0.0)
    z = jnp.sum(ex, axis=1, keepdims=True)
    z = jnp.where(z == 0.0, 1.0, z)
    w = jnp.sqrt(ex / z)

    gs = []
    for c in range(16):
        acc = soh[0] * w1v_ref[0, c]
        for b in range(1, NB):
            acc = acc + soh[b] * w1v_ref[b, c]
        hv = acc * (1.0 / (1.0 + jnp.exp(-acc)))
        gs.append(w * hv)
    g = jnp.concatenate(gs, axis=1)

    o0_ref[0] = jnp.dot(g, vtab[:, :14], precision=HI)
    for s, u in enumerate((ux, uy, uz)):
        ut = jnp.concatenate([u] * 16, axis=1)
        o1_ref[0, s] = jnp.dot(g * ut, vtab[:, 14:20], precision=HI)


def _mha_body(y_ref, wq_ref, wk_ref, wv_ref, bq_ref, bk_ref, bv_ref,
              wot_ref, bo_ref, o_ref):
    h = pl.program_id(1)
    yb = y_ref[0]
    cdims = (((1,), (1,)), ((), ()))
    q = jax.lax.dot_general(yb, wq_ref[0], cdims, precision=HI) + bq_ref[0]
    k = jax.lax.dot_general(yb, wk_ref[0], cdims, precision=HI) + bk_ref[0]
    v = jax.lax.dot_general(yb, wv_ref[0], cdims, precision=HI) + bv_ref[0]
    s = jax.lax.dot_general(q, k, cdims, precision=HI) * 0.25
    s = s - jnp.max(s, axis=1, keepdims=True)
    p = jnp.exp(s)
    p = p / jnp.sum(p, axis=1, keepdims=True)
    oh = jnp.dot(p, v, precision=HI)
    contrib = jnp.dot(oh, wot_ref[0], precision=HI)

    @pl.when(h == 0)
    def _():
        o_ref[0] = contrib + bo_ref[...]

    @pl.when(h != 0)
    def _():
        o_ref[0] = o_ref[0] + contrib


def kernel(x, y, W_hq, W1k, W2k, W1v, W2v, W_dot, in_proj_w, in_proj_b,
           out_proj_w, out_proj_b):
    B, N, D = x.shape
    Fi = D - 3
    nrm = math.sqrt(Fi)
    s_q = 1.0 / (4.0 * Fi * math.sqrt(D * 12.0))
    s_v = 1.0 / (4.0 * nrm)

    clouds = jnp.concatenate([x, y], axis=0)
    NC = 2 * B
    pos = clouds[:, :, :3]
    f = clouds[:, :, 3:]
    fpad = jnp.pad(f, ((0, 0), (0, 0), (0, D - Fi)))
    ft = jnp.swapaxes(fpad, 1, 2)
    post = jnp.swapaxes(pos, 1, 2)

    wkr = W2k[:, :Fi * 12].reshape(16, Fi, 12).transpose(2, 0, 1)
    wkrt = jnp.pad(wkr.reshape(192, Fi), ((0, 0), (0, D - Fi)))
    v0 = W2v[:, :Fi * 14].reshape(16, Fi, 14) * s_v
    v1 = W2v[:, Fi * 14:].reshape(16, Fi, 6) * (s_v * math.sqrt(3.0))
    wv_all = jnp.pad(jnp.concatenate([v0, v1], axis=2),
                     ((0, 0), (0, D - Fi), (0, 0)))
    whq_pad = jnp.pad(W_hq, ((0, D - Fi), (0, 0)))

    akt, vtab, qws = pl.pallas_call(
        functools.partial(_prep_body, s_q=s_q),
        grid=(NC,),
        in_specs=[
            pl.BlockSpec((1, N, D), lambda b: (b, 0, 0)),
            pl.BlockSpec((1, D, N), lambda b: (b, 0, 0)),
            pl.BlockSpec((192, D), lambda b: (0, 0)),
            pl.BlockSpec((16, D, 20), lambda b: (0, 0, 0)),
            pl.BlockSpec((D, D), lambda b: (0, 0)),
            pl.BlockSpec((D, 12), lambda b: (0, 0)),
        ],
        out_specs=[
            pl.BlockSpec((1, 192, N), lambda b: (b, 0, 0)),
            pl.BlockSpec((1, 16, N, 20), lambda b: (b, 0, 0, 0)),
            pl.BlockSpec((1, N, 12), lambda b: (b, 0, 0)),
        ],
        out_shape=[
            jax.ShapeDtypeStruct((NC, 192, N), jnp.float32),
            jax.ShapeDtypeStruct((NC, 16, N, 20), jnp.float32),
            jax.ShapeDtypeStruct((NC, N, 12), jnp.float32),
        ],
    )(fpad, ft, wkrt, wv_all, whq_pad, W_dot)

    akt = akt.reshape(NC, 12, 16 * N)
    vtab = vtab.reshape(NC, 16 * N, 20)

    nchunk = N // _C
    o0, o1 = pl.pallas_call(
        _edge_body,
        grid=(NC, nchunk),
        in_specs=[
            pl.BlockSpec((1, 3, N), lambda b, j: (b, 0, 0)),
            pl.BlockSpec((1, _C, 3), lambda b, j: (b, j, 0)),
            pl.BlockSpec((1, _C, 12), lambda b, j: (b, j, 0)),
            pl.BlockSpec((1, 12, 16 * N), lambda b, j: (b, 0, 0)),
            pl.BlockSpec((1, 16 * N, 20), lambda b, j: (b, 0, 0)),
            pl.BlockSpec(memory_space=pltpu.SMEM),
            pl.BlockSpec(memory_space=pltpu.SMEM),
        ],
        out_specs=[
            pl.BlockSpec((1, _C, 14), lambda b, j: (b, j, 0)),
            pl.BlockSpec((1, 3, _C, 6), lambda b, j: (b, 0, j, 0)),
        ],
        out_shape=[
            jax.ShapeDtypeStruct((NC, N, 14), jnp.float32),
            jax.ShapeDtypeStruct((NC, 3, N, 6), jnp.float32),
        ],
    )(post, pos, qws, akt, vtab, W1k, W1v)

    o1r = jnp.swapaxes(o1, 1, 2).reshape(NC, N, 3, 6)
    o1r = jnp.swapaxes(o1r, 2, 3).reshape(NC, N, 18)
    esa_out = jnp.concatenate([o0, o1r], axis=-1)
    xi = esa_out[:B]
    yi = esa_out[B:]

    dh = D // NHEAD
    wq = in_proj_w[:D].reshape(NHEAD, dh, D)
    wk = in_proj_w[D:2 * D].reshape(NHEAD, dh, D)
    wv = in_proj_w[2 * D:].reshape(NHEAD, dh, D)
    bq = in_proj_b[:D].reshape(NHEAD, 1, dh)
    bk = in_proj_b[D:2 * D].reshape(NHEAD, 1, dh)
    bv = in_proj_b[2 * D:].reshape(NHEAD, 1, dh)
    wot = jnp.swapaxes(out_proj_w, 0, 1).reshape(NHEAD, dh, D)
    bo = out_proj_b.reshape(1, D)

    yii = pl.pallas_call(
        _mha_body,
        grid=(B, NHEAD),
        in_specs=[
            pl.BlockSpec((1, N, D), lambda b, h: (b, 0, 0)),
            pl.BlockSpec((1, dh, D), lambda b, h: (h, 0, 0)),
            pl.BlockSpec((1, dh, D), lambda b, h: (h, 0, 0)),
            pl.BlockSpec((1, dh, D), lambda b, h: (h, 0, 0)),
            pl.BlockSpec((1, 1, dh), lambda b, h: (h, 0, 0)),
            pl.BlockSpec((1, 1, dh), lambda b, h: (h, 0, 0)),
            pl.BlockSpec((1, 1, dh), lambda b, h: (h, 0, 0)),
            pl.BlockSpec((1, dh, D), lambda b, h: (h, 0, 0)),
            pl.BlockSpec((1, D), lambda b, h: (0, 0)),
        ],
        out_specs=pl.BlockSpec((1, N, D), lambda b, h: (b, 0, 0)),
        out_shape=jax.ShapeDtypeStruct((B, N, D), jnp.float32),
    )(y, wq, wk, wv, bq, bk, bv, wot, bo)

    return jnp.concatenate([xi, yi, yii], axis=-1)


# speedup vs baseline: 162.8596x; 2.1667x over previous
import functools
import math

import jax
import jax.numpy as jnp
import numpy as np
from jax.experimental import pallas as pl
from jax.experimental.pallas import tpu as pltpu

RADIUS = 2.0
NB = 10
NHEAD = 8
HI = jax.lax.Precision.HIGHEST
MED = jax.lax.Precision.DEFAULT

_C = 64
_N = 1024
_SOH_STEP = RADIUS / (NB + 1)
_SOH_CONST = 1.14136 * float(np.exp(2.0))


def _sus(x):
    return jnp.where(x > 0.0, jnp.exp(-1.0 / jnp.where(x > 0.0, x, 1.0)), 0.0)


def _prep_body(f_ref, ft_ref, wkrt_ref, wv_ref, whq_ref, wdot_ref,
               akt_ref, vtab_ref, qws_ref, *, s_q):
    f = f_ref[0]
    ft = ft_ref[0]
    akt_ref[0] = jnp.dot(wkrt_ref[...], ft, precision=HI)
    for c in range(16):
        vtab_ref[0, c] = jnp.dot(f, wv_ref[c], precision=HI)
    wqd = jnp.dot(whq_ref[...], wdot_ref[...], precision=HI)
    qws_ref[0] = jnp.dot(f, wqd, precision=HI) * s_q


def _edge_body(post_ref, posd_ref, qws_ref, akt_ref, vtab_ref,
               w1k_ref, w1v_ref, o0_ref, o1_ref):
    j = pl.program_id(1)
    post = post_ref[0]
    posd = posd_ref[0]
    qws = qws_ref[0]
    akt = akt_ref[0]
    vtab = vtab_ref[0]

    p2 = jnp.dot(qws, akt, precision=HI)

    dx = post[0:1, :] - posd[:, 0:1]
    dy = post[1:2, :] - posd[:, 1:2]
    dz = post[2:3, :] - posd[:, 2:3]
    len2 = dx * dx + dy * dy + dz * dz
    elen = jnp.sqrt(len2)
    inv = jax.lax.rsqrt(jnp.where(len2 > 0.0, len2, 1.0))
    ux = dx * inv
    uy = dy * inv
    uz = dz * inv

    src = jax.lax.broadcasted_iota(jnp.int32, (_C, _N), 1)
    dst = jax.lax.broadcasted_iota(jnp.int32, (_C, _N), 0) + j * _C
    mask = (elen < RADIUS) & (src != dst)
    cutoff = _sus(10.0 - 5.0 * elen)

    u = elen * (1.0 / _SOH_STEP)
    soh = []
    for b in range(NB):
        t1 = u - b
        t2 = (b + 2) - u
        inside = (t1 > 0.0) & (t2 > 0.0)
        arg = (1.0 / jnp.where(inside, t1, 1.0)
               + 1.0 / jnp.where(inside, t2, 1.0))
        soh.append(jnp.where(inside, _SOH_CONST * jnp.exp(-arg), 0.0))

    dotqk = jnp.zeros((_C, _N), jnp.float32)
    for c in range(16):
        acc = soh[0] * w1k_ref[0, c]
        for b in range(1, NB):
            acc = acc + soh[b] * w1k_ref[b, c]
        hk = acc * (1.0 / (1.0 + jnp.exp(-acc)))
        dotqk = dotqk + hk * p2[:, c * _N:(c + 1) * _N]

    ex = jnp.where(mask, cutoff * jnp.exp(dotqk), 0.0)
    z = jnp.sum(ex, axis=1, keepdims=True)
    z = jnp.where(z == 0.0, 1.0, z)
    w = jnp.sqrt(ex / z)

    gs = []
    for c in range(16):
        acc = soh[0] * w1v_ref[0, c]
        for b in range(1, NB):
            acc = acc + soh[b] * w1v_ref[b, c]
        hv = acc * (1.0 / (1.0 + jnp.exp(-acc)))
        gs.append(w * hv)
    g = jnp.concatenate(gs, axis=1)

    o0_ref[0] = jnp.dot(g, vtab[:, :14], precision=MED)
    for s, u in enumerate((ux, uy, uz)):
        ut = jnp.concatenate([u] * 16, axis=1)
        o1_ref[0, s] = jnp.dot(g * ut, vtab[:, 14:20], precision=MED)


def _mha_body(y_ref, wq_ref, wk_ref, wv_ref, bq_ref, bk_ref, bv_ref,
              wot_ref, bo_ref, o_ref):
    h = pl.program_id(1)
    yb = y_ref[0]
    cdims = (((1,), (1,)), ((), ()))
    q = jax.lax.dot_general(yb, wq_ref[0], cdims, precision=HI) + bq_ref[0]
    k = jax.lax.dot_general(yb, wk_ref[0], cdims, precision=HI) + bk_ref[0]
    v = jax.lax.dot_general(yb, wv_ref[0], cdims, precision=HI) + bv_ref[0]
    s = jax.lax.dot_general(q, k, cdims, precision=MED) * 0.25
    s = s - jnp.max(s, axis=1, keepdims=True)
    p = jnp.exp(s)
    p = p / jnp.sum(p, axis=1, keepdims=True)
    oh = jnp.dot(p, v, precision=MED)
    contrib = jnp.dot(oh, wot_ref[0], precision=MED)

    @pl.when(h == 0)
    def _():
        o_ref[0] = contrib + bo_ref[...]

    @pl.when(h != 0)
    def _():
        o_ref[0] = o_ref[0] + contrib


def kernel(x, y, W_hq, W1k, W2k, W1v, W2v, W_dot, in_proj_w, in_proj_b,
           out_proj_w, out_proj_b):
    B, N, D = x.shape
    Fi = D - 3
    nrm = math.sqrt(Fi)
    s_q = 1.0 / (4.0 * Fi * math.sqrt(D * 12.0))
    s_v = 1.0 / (4.0 * nrm)

    clouds = jnp.concatenate([x, y], axis=0)
    NC = 2 * B
    pos = clouds[:, :, :3]
    f = clouds[:, :, 3:]
    fpad = jnp.pad(f, ((0, 0), (0, 0), (0, D - Fi)))
    ft = jnp.swapaxes(fpad, 1, 2)
    post = jnp.swapaxes(pos, 1, 2)

    wkr = W2k[:, :Fi * 12].reshape(16, Fi, 12).transpose(2, 0, 1)
    wkrt = jnp.pad(wkr.reshape(192, Fi), ((0, 0), (0, D - Fi)))
    v0 = W2v[:, :Fi * 14].reshape(16, Fi, 14) * s_v
    v1 = W2v[:, Fi * 14:].reshape(16, Fi, 6) * (s_v * math.sqrt(3.0))
    wv_all = jnp.pad(jnp.concatenate([v0, v1], axis=2),
                     ((0, 0), (0, D - Fi), (0, 0)))
    whq_pad = jnp.pad(W_hq, ((0, D - Fi), (0, 0)))

    akt, vtab, qws = pl.pallas_call(
        functools.partial(_prep_body, s_q=s_q),
        grid=(NC,),
        in_specs=[
            pl.BlockSpec((1, N, D), lambda b: (b, 0, 0)),
            pl.BlockSpec((1, D, N), lambda b: (b, 0, 0)),
            pl.BlockSpec((192, D), lambda b: (0, 0)),
            pl.BlockSpec((16, D, 20), lambda b: (0, 0, 0)),
            pl.BlockSpec((D, D), lambda b: (0, 0)),
            pl.BlockSpec((D, 12), lambda b: (0, 0)),
        ],
        out_specs=[
            pl.BlockSpec((1, 192, N), lambda b: (b, 0, 0)),
            pl.BlockSpec((1, 16, N, 20), lambda b: (b, 0, 0, 0)),
            pl.BlockSpec((1, N, 12), lambda b: (b, 0, 0)),
        ],
        out_shape=[
            jax.ShapeDtypeStruct((NC, 192, N), jnp.float32),
            jax.ShapeDtypeStruct((NC, 16, N, 20), jnp.float32),
            jax.ShapeDtypeStruct((NC, N, 12), jnp.float32),
        ],
    )(fpad, ft, wkrt, wv_all, whq_pad, W_dot)

    akt = akt.reshape(NC, 12, 16 * N)
    vtab = vtab.reshape(NC, 16 * N, 20)

    nchunk = N // _C
    o0, o1 = pl.pallas_call(
        _edge_body,
        grid=(NC, nchunk),
        in_specs=[
            pl.BlockSpec((1, 3, N), lambda b, j: (b, 0, 0)),
            pl.BlockSpec((1, _C, 3), lambda b, j: (b, j, 0)),
            pl.BlockSpec((1, _C, 12), lambda b, j: (b, j, 0)),
            pl.BlockSpec((1, 12, 16 * N), lambda b, j: (b, 0, 0)),
            pl.BlockSpec((1, 16 * N, 20), lambda b, j: (b, 0, 0)),
            pl.BlockSpec(memory_space=pltpu.SMEM),
            pl.BlockSpec(memory_space=pltpu.SMEM),
        ],
        out_specs=[
            pl.BlockSpec((1, _C, 14), lambda b, j: (b, j, 0)),
            pl.BlockSpec((1, 3, _C, 6), lambda b, j: (b, 0, j, 0)),
        ],
        out_shape=[
            jax.ShapeDtypeStruct((NC, N, 14), jnp.float32),
            jax.ShapeDtypeStruct((NC, 3, N, 6), jnp.float32),
        ],
    )(post, pos, qws, akt, vtab, W1k, W1v)

    o1r = jnp.swapaxes(o1, 1, 2).reshape(NC, N, 3, 6)
    o1r = jnp.swapaxes(o1r, 2, 3).reshape(NC, N, 18)
    esa_out = jnp.concatenate([o0, o1r], axis=-1)
    xi = esa_out[:B]
    yi = esa_out[B:]

    dh = D // NHEAD
    wq = in_proj_w[:D].reshape(NHEAD, dh, D)
    wk = in_proj_w[D:2 * D].reshape(NHEAD, dh, D)
    wv = in_proj_w[2 * D:].reshape(NHEAD, dh, D)
    bq = in_proj_b[:D].reshape(NHEAD, 1, dh)
    bk = in_proj_b[D:2 * D].reshape(NHEAD, 1, dh)
    bv = in_proj_b[2 * D:].reshape(NHEAD, 1, dh)
    wot = jnp.swapaxes(out_proj_w, 0, 1).reshape(NHEAD, dh, D)
    bo = out_proj_b.reshape(1, D)

    yii = pl.pallas_call(
        _mha_body,
        grid=(B, NHEAD),
        in_specs=[
            pl.BlockSpec((1, N, D), lambda b, h: (b, 0, 0)),
            pl.BlockSpec((1, dh, D), lambda b, h: (h, 0, 0)),
            pl.BlockSpec((1, dh, D), lambda b, h: (h, 0, 0)),
            pl.BlockSpec((1, dh, D), lambda b, h: (h, 0, 0)),
            pl.BlockSpec((1, 1, dh), lambda b, h: (h, 0, 0)),
            pl.BlockSpec((1, 1, dh), lambda b, h: (h, 0, 0)),
            pl.BlockSpec((1, 1, dh), lambda b, h: (h, 0, 0)),
            pl.BlockSpec((1, dh, D), lambda b, h: (h, 0, 0)),
            pl.BlockSpec((1, D), lambda b, h: (0, 0)),
        ],
        out_specs=pl.BlockSpec((1, N, D), lambda b, h: (b, 0, 0)),
        out_shape=jax.ShapeDtypeStruct((B, N, D), jnp.float32),
    )(y, wq, wk, wv, bq, bk, bv, wot, bo)

    return jnp.concatenate([xi, yi, yii], axis=-1)


# speedup vs baseline: 165.2532x; 1.0147x over previous
import functools
import math

import jax
import jax.numpy as jnp
import numpy as np
from jax.experimental import pallas as pl
from jax.experimental.pallas import tpu as pltpu

RADIUS = 2.0
NB = 10
NHEAD = 8
HI = jax.lax.Precision.HIGHEST
MED = jax.lax.Precision.DEFAULT

_C = 64
_N = 1024
_SOH_STEP = RADIUS / (NB + 1)
_SOH_CONST = 1.14136 * float(np.exp(2.0))


def _sus(x):
    return jnp.where(x > 0.0, jnp.exp(-1.0 / jnp.where(x > 0.0, x, 1.0)), 0.0)


def _prep_body(f_ref, ft_ref, wkrt_ref, wv_ref, whq_ref, wdot_ref,
               akt_ref, vtab_ref, qws_ref, *, s_q):
    f = f_ref[0]
    ft = ft_ref[0]
    akt_ref[0] = jnp.dot(wkrt_ref[...], ft, precision=HI)
    for c in range(16):
        vtab_ref[0, c] = jnp.dot(f, wv_ref[c], precision=HI)
    wqd = jnp.dot(whq_ref[...], wdot_ref[...], precision=HI)
    qws_ref[0] = jnp.dot(f, wqd, precision=HI) * s_q


def _edge_body(post_ref, posd_ref, qws_ref, akt_ref, vtab_ref,
               w1k_ref, w1v_ref, o0_ref, o1_ref):
    j = pl.program_id(1)
    post = post_ref[0]
    posd = posd_ref[0]
    qws = qws_ref[0]
    akt = akt_ref[0]
    vtab = vtab_ref[0]

    p2 = jnp.dot(qws, akt, precision=HI)

    dx = post[0:1, :] - posd[:, 0:1]
    dy = post[1:2, :] - posd[:, 1:2]
    dz = post[2:3, :] - posd[:, 2:3]
    len2 = dx * dx + dy * dy + dz * dz
    elen = jnp.sqrt(len2)
    inv = jax.lax.rsqrt(jnp.where(len2 > 0.0, len2, 1.0))
    ux = dx * inv
    uy = dy * inv
    uz = dz * inv

    src = jax.lax.broadcasted_iota(jnp.int32, (_C, _N), 1)
    dst = jax.lax.broadcasted_iota(jnp.int32, (_C, _N), 0) + j * _C
    mask = (elen < RADIUS) & (src != dst)
    cutoff = _sus(10.0 - 5.0 * elen)

    u = elen * (1.0 / _SOH_STEP)
    soh = []
    for b in range(NB):
        t1 = u - b
        t2 = (b + 2) - u
        inside = (t1 > 0.0) & (t2 > 0.0)
        arg = 1.0 / t1 + 1.0 / t2
        soh.append(jnp.where(inside, _SOH_CONST * jnp.exp(-arg), 0.0))

    dotqk = jnp.zeros((_C, _N), jnp.float32)
    hvs = []
    for c in range(16):
        acck = soh[0] * w1k_ref[0, c]
        accv = soh[0] * w1v_ref[0, c]
        for b in range(1, NB):
            acck = acck + soh[b] * w1k_ref[b, c]
            accv = accv + soh[b] * w1v_ref[b, c]
        hk = acck * (1.0 / (1.0 + jnp.exp(-acck)))
        hvs.append(accv * (1.0 / (1.0 + jnp.exp(-accv))))
        dotqk = dotqk + hk * p2[:, c * _N:(c + 1) * _N]

    ex = jnp.where(mask, cutoff * jnp.exp(dotqk), 0.0)
    z = jnp.sum(ex, axis=1, keepdims=True)
    z = jnp.where(z == 0.0, 1.0, z)
    w = jnp.sqrt(ex / z)

    g = jnp.concatenate([w * hv for hv in hvs], axis=1)

    o0_ref[0] = jnp.dot(g, vtab[:, :14], precision=MED)
    for s, u in enumerate((ux, uy, uz)):
        ut = jnp.concatenate([u] * 16, axis=1)
        o1_ref[0, s] = jnp.dot(g * ut, vtab[:, 14:20], precision=MED)


def _mha_body(y_ref, wq_ref, wk_ref, wv_ref, bq_ref, bk_ref, bv_ref,
              wot_ref, bo_ref, o_ref):
    h = pl.program_id(1)
    yb = y_ref[0]
    cdims = (((1,), (1,)), ((), ()))
    q = jax.lax.dot_general(yb, wq_ref[0], cdims, precision=HI) + bq_ref[0]
    k = jax.lax.dot_general(yb, wk_ref[0], cdims, precision=HI) + bk_ref[0]
    v = jax.lax.dot_general(yb, wv_ref[0], cdims, precision=HI) + bv_ref[0]
    s = jax.lax.dot_general(q, k, cdims, precision=MED) * 0.25
    s = s - jnp.max(s, axis=1, keepdims=True)
    p = jnp.exp(s)
    p = p / jnp.sum(p, axis=1, keepdims=True)
    oh = jnp.dot(p, v, precision=MED)
    contrib = jnp.dot(oh, wot_ref[0], precision=MED)

    @pl.when(h == 0)
    def _():
        o_ref[0] = contrib + bo_ref[...]

    @pl.when(h != 0)
    def _():
        o_ref[0] = o_ref[0] + contrib


def kernel(x, y, W_hq, W1k, W2k, W1v, W2v, W_dot, in_proj_w, in_proj_b,
           out_proj_w, out_proj_b):
    B, N, D = x.shape
    Fi = D - 3
    nrm = math.sqrt(Fi)
    s_q = 1.0 / (4.0 * Fi * math.sqrt(D * 12.0))
    s_v = 1.0 / (4.0 * nrm)

    clouds = jnp.concatenate([x, y], axis=0)
    NC = 2 * B
    pos = clouds[:, :, :3]
    f = clouds[:, :, 3:]
    fpad = jnp.pad(f, ((0, 0), (0, 0), (0, D - Fi)))
    ft = jnp.swapaxes(fpad, 1, 2)
    post = jnp.swapaxes(pos, 1, 2)

    wkr = W2k[:, :Fi * 12].reshape(16, Fi, 12).transpose(2, 0, 1)
    wkrt = jnp.pad(wkr.reshape(192, Fi), ((0, 0), (0, D - Fi)))
    v0 = W2v[:, :Fi * 14].reshape(16, Fi, 14) * s_v
    v1 = W2v[:, Fi * 14:].reshape(16, Fi, 6) * (s_v * math.sqrt(3.0))
    wv_all = jnp.pad(jnp.concatenate([v0, v1], axis=2),
                     ((0, 0), (0, D - Fi), (0, 0)))
    whq_pad = jnp.pad(W_hq, ((0, D - Fi), (0, 0)))

    akt, vtab, qws = pl.pallas_call(
        functools.partial(_prep_body, s_q=s_q),
        grid=(NC,),
        in_specs=[
            pl.BlockSpec((1, N, D), lambda b: (b, 0, 0)),
            pl.BlockSpec((1, D, N), lambda b: (b, 0, 0)),
            pl.BlockSpec((192, D), lambda b: (0, 0)),
            pl.BlockSpec((16, D, 20), lambda b: (0, 0, 0)),
            pl.BlockSpec((D, D), lambda b: (0, 0)),
            pl.BlockSpec((D, 12), lambda b: (0, 0)),
        ],
        out_specs=[
            pl.BlockSpec((1, 192, N), lambda b: (b, 0, 0)),
            pl.BlockSpec((1, 16, N, 20), lambda b: (b, 0, 0, 0)),
            pl.BlockSpec((1, N, 12), lambda b: (b, 0, 0)),
        ],
        out_shape=[
            jax.ShapeDtypeStruct((NC, 192, N), jnp.float32),
            jax.ShapeDtypeStruct((NC, 16, N, 20), jnp.float32),
            jax.ShapeDtypeStruct((NC, N, 12), jnp.float32),
        ],
    )(fpad, ft, wkrt, wv_all, whq_pad, W_dot)

    akt = akt.reshape(NC, 12, 16 * N)
    vtab = vtab.reshape(NC, 16 * N, 20)

    nchunk = N // _C
    o0, o1 = pl.pallas_call(
        _edge_body,
        grid=(NC, nchunk),
        in_specs=[
            pl.BlockSpec((1, 3, N), lambda b, j: (b, 0, 0)),
            pl.BlockSpec((1, _C, 3), lambda b, j: (b, j, 0)),
            pl.BlockSpec((1, _C, 12), lambda b, j: (b, j, 0)),
            pl.BlockSpec((1, 12, 16 * N), lambda b, j: (b, 0, 0)),
            pl.BlockSpec((1, 16 * N, 20), lambda b, j: (b, 0, 0)),
            pl.BlockSpec(memory_space=pltpu.SMEM),
            pl.BlockSpec(memory_space=pltpu.SMEM),
        ],
        out_specs=[
            pl.BlockSpec((1, _C, 14), lambda b, j: (b, j, 0)),
            pl.BlockSpec((1, 3, _C, 6), lambda b, j: (b, 0, j, 0)),
        ],
        out_shape=[
            jax.ShapeDtypeStruct((NC, N, 14), jnp.float32),
            jax.ShapeDtypeStruct((NC, 3, N, 6), jnp.float32),
        ],
    )(post, pos, qws, akt, vtab, W1k, W1v)

    o1r = jnp.swapaxes(o1, 1, 2).reshape(NC, N, 3, 6)
    o1r = jnp.swapaxes(o1r, 2, 3).reshape(NC, N, 18)
    esa_out = jnp.concatenate([o0, o1r], axis=-1)
    xi = esa_out[:B]
    yi = esa_out[B:]

    dh = D // NHEAD
    wq = in_proj_w[:D].reshape(NHEAD, dh, D)
    wk = in_proj_w[D:2 * D].reshape(NHEAD, dh, D)
    wv = in_proj_w[2 * D:].reshape(NHEAD, dh, D)
    bq = in_proj_b[:D].reshape(NHEAD, 1, dh)
    bk = in_proj_b[D:2 * D].reshape(NHEAD, 1, dh)
    bv = in_proj_b[2 * D:].reshape(NHEAD, 1, dh)
    wot = jnp.swapaxes(out_proj_w, 0, 1).reshape(NHEAD, dh, D)
    bo = out_proj_b.reshape(1, D)

    yii = pl.pallas_call(
        _mha_body,
        grid=(B, NHEAD),
        in_specs=[
            pl.BlockSpec((1, N, D), lambda b, h: (b, 0, 0)),
            pl.BlockSpec((1, dh, D), lambda b, h: (h, 0, 0)),
            pl.BlockSpec((1, dh, D), lambda b, h: (h, 0, 0)),
            pl.BlockSpec((1, dh, D), lambda b, h: (h, 0, 0)),
            pl.BlockSpec((1, 1, dh), lambda b, h: (h, 0, 0)),
            pl.BlockSpec((1, 1, dh), lambda b, h: (h, 0, 0)),
            pl.BlockSpec((1, 1, dh), lambda b, h: (h, 0, 0)),
            pl.BlockSpec((1, dh, D), lambda b, h: (h, 0, 0)),
            pl.BlockSpec((1, D), lambda b, h: (0, 0)),
        ],
        out_specs=pl.BlockSpec((1, N, D), lambda b, h: (b, 0, 0)),
        out_shape=jax.ShapeDtypeStruct((B, N, D), jnp.float32),
    )(y, wq, wk, wv, bq, bk, bv, wot, bo)

    return jnp.concatenate([xi, yi, yii], axis=-1)


# speedup vs baseline: 165.9390x; 1.0042x over previous
import functools
import math

import jax
import jax.numpy as jnp
import numpy as np
from jax.experimental import pallas as pl
from jax.experimental.pallas import tpu as pltpu

RADIUS = 2.0
NB = 10
NHEAD = 8
HI = jax.lax.Precision.HIGHEST
MED = jax.lax.Precision.DEFAULT

_C = 64
_N = 1024
_SOH_STEP = RADIUS / (NB + 1)
_SOH_CONST = 1.14136 * float(np.exp(2.0))


def _sus(x):
    return jnp.where(x > 0.0, jnp.exp(-1.0 / jnp.where(x > 0.0, x, 1.0)), 0.0)


def _prep_body(f_ref, wkrt_ref, wv_ref, whq_ref, wdot_ref,
               akt_ref, vtab_ref, qws_ref, *, s_q):
    f = f_ref[0]
    cdims = (((1,), (1,)), ((), ()))
    akt_ref[0] = jax.lax.dot_general(
        wkrt_ref[...], f, cdims, precision=HI)
    for c in range(16):
        vtab_ref[0, c] = jnp.dot(f, wv_ref[c], precision=HI)
    wqd = jnp.dot(whq_ref[...], wdot_ref[...], precision=HI)
    qws_ref[0] = jnp.dot(f, wqd, precision=HI) * s_q


def _edge_body(post_ref, posd_ref, qws_ref, akt_ref, vtab_ref,
               w1k_ref, w1v_ref, o0_ref, o1_ref):
    j = pl.program_id(1)
    post = post_ref[0]
    posd = posd_ref[0]
    qws = qws_ref[0]
    akt = akt_ref[0]
    vtab = vtab_ref[0]

    p2 = jnp.dot(qws, akt, precision=HI)

    dx = post[0:1, :] - posd[:, 0:1]
    dy = post[1:2, :] - posd[:, 1:2]
    dz = post[2:3, :] - posd[:, 2:3]
    len2 = dx * dx + dy * dy + dz * dz
    elen = jnp.sqrt(len2)
    inv = jax.lax.rsqrt(jnp.where(len2 > 0.0, len2, 1.0))
    ux = dx * inv
    uy = dy * inv
    uz = dz * inv

    src = jax.lax.broadcasted_iota(jnp.int32, (_C, _N), 1)
    dst = jax.lax.broadcasted_iota(jnp.int32, (_C, _N), 0) + j * _C
    mask = (elen < RADIUS) & (src != dst)
    cutoff = _sus(10.0 - 5.0 * elen)

    u = elen * (1.0 / _SOH_STEP)
    soh = []
    for b in range(NB):
        t1 = u - b
        t2 = (b + 2) - u
        inside = (t1 > 0.0) & (t2 > 0.0)
        arg = 1.0 / t1 + 1.0 / t2
        soh.append(jnp.where(inside, _SOH_CONST * jnp.exp(-arg), 0.0))

    dotqk = jnp.zeros((_C, _N), jnp.float32)
    hvs = []
    for c in range(16):
        acck = soh[0] * w1k_ref[0, c]
        accv = soh[0] * w1v_ref[0, c]
        for b in range(1, NB):
            acck = acck + soh[b] * w1k_ref[b, c]
            accv = accv + soh[b] * w1v_ref[b, c]
        hk = acck * (1.0 / (1.0 + jnp.exp(-acck)))
        hvs.append(accv * (1.0 / (1.0 + jnp.exp(-accv))))
        dotqk = dotqk + hk * p2[:, c * _N:(c + 1) * _N]

    ex = jnp.where(mask, cutoff * jnp.exp(dotqk), 0.0)
    z = jnp.sum(ex, axis=1, keepdims=True)
    z = jnp.where(z == 0.0, 1.0, z)
    w = jnp.sqrt(ex / z)

    g = jnp.concatenate([w * hv for hv in hvs], axis=1)

    o0_ref[0] = jnp.dot(g, vtab[:, :14], precision=MED)
    for s, u in enumerate((ux, uy, uz)):
        ut = jnp.concatenate([u] * 16, axis=1)
        o1_ref[0, s] = jnp.dot(g * ut, vtab[:, 14:20], precision=MED)


def _mha_body(y_ref, wq_ref, wk_ref, wv_ref, bq_ref, bk_ref, bv_ref,
              wot_ref, bo_ref, o_ref):
    h = pl.program_id(1)
    yb = y_ref[0]
    cdims = (((1,), (1,)), ((), ()))
    q = jax.lax.dot_general(yb, wq_ref[0], cdims, precision=HI) + bq_ref[0]
    k = jax.lax.dot_general(yb, wk_ref[0], cdims, precision=HI) + bk_ref[0]
    v = jax.lax.dot_general(yb, wv_ref[0], cdims, precision=HI) + bv_ref[0]
    s = jax.lax.dot_general(q, k, cdims, precision=MED) * 0.25
    s = s - jnp.max(s, axis=1, keepdims=True)
    p = jnp.exp(s)
    p = p / jnp.sum(p, axis=1, keepdims=True)
    oh = jnp.dot(p, v, precision=MED)
    contrib = jnp.dot(oh, wot_ref[0], precision=MED)

    @pl.when(h == 0)
    def _():
        o_ref[0] = contrib + bo_ref[...]

    @pl.when(h != 0)
    def _():
        o_ref[0] = o_ref[0] + contrib


def kernel(x, y, W_hq, W1k, W2k, W1v, W2v, W_dot, in_proj_w, in_proj_b,
           out_proj_w, out_proj_b):
    B, N, D = x.shape
    Fi = D - 3
    nrm = math.sqrt(Fi)
    s_q = 1.0 / (4.0 * Fi * math.sqrt(D * 12.0))
    s_v = 1.0 / (4.0 * nrm)

    clouds = jnp.concatenate([x, y], axis=0)
    NC = 2 * B
    pos = clouds[:, :, :3]
    f = clouds[:, :, 3:]
    fpad = jnp.pad(f, ((0, 0), (0, 0), (0, D - Fi)))
    post = jnp.swapaxes(pos, 1, 2)

    wkr = W2k[:, :Fi * 12].reshape(16, Fi, 12).transpose(2, 0, 1)
    wkrt = jnp.pad(wkr.reshape(192, Fi), ((0, 0), (0, D - Fi)))
    v0 = W2v[:, :Fi * 14].reshape(16, Fi, 14) * s_v
    v1 = W2v[:, Fi * 14:].reshape(16, Fi, 6) * (s_v * math.sqrt(3.0))
    wv_all = jnp.pad(jnp.concatenate([v0, v1], axis=2),
                     ((0, 0), (0, D - Fi), (0, 0)))
    whq_pad = jnp.pad(W_hq, ((0, D - Fi), (0, 0)))

    akt, vtab, qws = pl.pallas_call(
        functools.partial(_prep_body, s_q=s_q),
        grid=(NC,),
        in_specs=[
            pl.BlockSpec((1, N, D), lambda b: (b, 0, 0)),
            pl.BlockSpec((192, D), lambda b: (0, 0)),
            pl.BlockSpec((16, D, 20), lambda b: (0, 0, 0)),
            pl.BlockSpec((D, D), lambda b: (0, 0)),
            pl.BlockSpec((D, 12), lambda b: (0, 0)),
        ],
        out_specs=[
            pl.BlockSpec((1, 192, N), lambda b: (b, 0, 0)),
            pl.BlockSpec((1, 16, N, 20), lambda b: (b, 0, 0, 0)),
            pl.BlockSpec((1, N, 12), lambda b: (b, 0, 0)),
        ],
        out_shape=[
            jax.ShapeDtypeStruct((NC, 192, N), jnp.float32),
            jax.ShapeDtypeStruct((NC, 16, N, 20), jnp.float32),
            jax.ShapeDtypeStruct((NC, N, 12), jnp.float32),
        ],
    )(fpad, wkrt, wv_all, whq_pad, W_dot)

    akt = akt.reshape(NC, 12, 16 * N)
    vtab = vtab.reshape(NC, 16 * N, 20)

    nchunk = N // _C
    o0, o1 = pl.pallas_call(
        _edge_body,
        grid=(NC, nchunk),
        in_specs=[
            pl.BlockSpec((1, 3, N), lambda b, j: (b, 0, 0)),
            pl.BlockSpec((1, _C, 3), lambda b, j: (b, j, 0)),
            pl.BlockSpec((1, _C, 12), lambda b, j: (b, j, 0)),
            pl.BlockSpec((1, 12, 16 * N), lambda b, j: (b, 0, 0)),
            pl.BlockSpec((1, 16 * N, 20), lambda b, j: (b, 0, 0)),
            pl.BlockSpec(memory_space=pltpu.SMEM),
            pl.BlockSpec(memory_space=pltpu.SMEM),
        ],
        out_specs=[
            pl.BlockSpec((1, _C, 14), lambda b, j: (b, j, 0)),
            pl.BlockSpec((1, 3, _C, 6), lambda b, j: (b, 0, j, 0)),
        ],
        out_shape=[
            jax.ShapeDtypeStruct((NC, N, 14), jnp.float32),
            jax.ShapeDtypeStruct((NC, 3, N, 6), jnp.float32),
        ],
    )(post, pos, qws, akt, vtab, W1k, W1v)

    o1r = jnp.swapaxes(o1, 1, 2).reshape(NC, N, 3, 6)
    o1r = jnp.swapaxes(o1r, 2, 3).reshape(NC, N, 18)
    esa_out = jnp.concatenate([o0, o1r], axis=-1)
    xi = esa_out[:B]
    yi = esa_out[B:]

    dh = D // NHEAD
    wq = in_proj_w[:D].reshape(NHEAD, dh, D)
    wk = in_proj_w[D:2 * D].reshape(NHEAD, dh, D)
    wv = in_proj_w[2 * D:].reshape(NHEAD, dh, D)
    bq = in_proj_b[:D].reshape(NHEAD, 1, dh)
    bk = in_proj_b[D:2 * D].reshape(NHEAD, 1, dh)
    bv = in_proj_b[2 * D:].reshape(NHEAD, 1, dh)
    wot = jnp.swapaxes(out_proj_w, 0, 1).reshape(NHEAD, dh, D)
    bo = out_proj_b.reshape(1, D)

    yii = pl.pallas_call(
        _mha_body,
        grid=(B, NHEAD),
        in_specs=[
            pl.BlockSpec((1, N, D), lambda b, h: (b, 0, 0)),
            pl.BlockSpec((1, dh, D), lambda b, h: (h, 0, 0)),
            pl.BlockSpec((1, dh, D), lambda b, h: (h, 0, 0)),
            pl.BlockSpec((1, dh, D), lambda b, h: (h, 0, 0)),
            pl.BlockSpec((1, 1, dh), lambda b, h: (h, 0, 0)),
            pl.BlockSpec((1, 1, dh), lambda b, h: (h, 0, 0)),
            pl.BlockSpec((1, 1, dh), lambda b, h: (h, 0, 0)),
            pl.BlockSpec((1, dh, D), lambda b, h: (h, 0, 0)),
            pl.BlockSpec((1, D), lambda b, h: (0, 0)),
        ],
        out_specs=pl.BlockSpec((1, N, D), lambda b, h: (b, 0, 0)),
        out_shape=jax.ShapeDtypeStruct((B, N, D), jnp.float32),
    )(y, wq, wk, wv, bq, bk, bv, wot, bo)

    return jnp.concatenate([xi, yi, yii], axis=-1)


# speedup vs baseline: 167.8370x; 1.0114x over previous
import functools
import math

import jax
import jax.numpy as jnp
import numpy as np
from jax.experimental import pallas as pl
from jax.experimental.pallas import tpu as pltpu

RADIUS = 2.0
NB = 10
NHEAD = 8
HI = jax.lax.Precision.HIGHEST
MED = jax.lax.Precision.DEFAULT

_C = 128
_N = 1024
_SOH_STEP = RADIUS / (NB + 1)
_SOH_CONST = 1.14136 * float(np.exp(2.0))


def _sus(x):
    return jnp.where(x > 0.0, jnp.exp(-1.0 / jnp.where(x > 0.0, x, 1.0)), 0.0)


def _prep_body(f_ref, wkrt_ref, wv_ref, whq_ref, wdot_ref,
               akt_ref, vtab_ref, qws_ref, *, s_q):
    f = f_ref[0]
    cdims = (((1,), (1,)), ((), ()))
    akt_ref[0] = jax.lax.dot_general(
        wkrt_ref[...], f, cdims, precision=HI)
    for c in range(16):
        vtab_ref[0, c] = jnp.dot(f, wv_ref[c], precision=HI)
    wqd = jnp.dot(whq_ref[...], wdot_ref[...], precision=HI)
    qws_ref[0] = jnp.dot(f, wqd, precision=HI) * s_q


def _edge_body(post_ref, posd_ref, qws_ref, akt_ref, vtab_ref,
               w1k_ref, w1v_ref, o0_ref, o1_ref):
    j = pl.program_id(1)
    post = post_ref[0]
    posd = posd_ref[0]
    qws = qws_ref[0]
    akt = akt_ref[0]
    vtab = vtab_ref[0]

    p2 = jnp.dot(qws, akt, precision=HI)

    dx = post[0:1, :] - posd[:, 0:1]
    dy = post[1:2, :] - posd[:, 1:2]
    dz = post[2:3, :] - posd[:, 2:3]
    len2 = dx * dx + dy * dy + dz * dz
    inv = jax.lax.rsqrt(jnp.where(len2 > 0.0, len2, 1.0))
    elen = len2 * inv
    ux = dx * inv
    uy = dy * inv
    uz = dz * inv

    src = jax.lax.broadcasted_iota(jnp.int32, (_C, _N), 1)
    dst = jax.lax.broadcasted_iota(jnp.int32, (_C, _N), 0) + j * _C
    mask = (elen < RADIUS) & (src != dst)
    cutoff = _sus(10.0 - 5.0 * elen)

    u = elen * (1.0 / _SOH_STEP)
    soh = []
    for b in range(NB):
        t1 = u - b
        t2 = (b + 2) - u
        inside = (t1 > 0.0) & (t2 > 0.0)
        arg = 1.0 / t1 + 1.0 / t2
        soh.append(jnp.where(inside, _SOH_CONST * jnp.exp(-arg), 0.0))

    dotqk = jnp.zeros((_C, _N), jnp.float32)
    hvs = []
    for c in range(16):
        acck = soh[0] * w1k_ref[0, c]
        accv = soh[0] * w1v_ref[0, c]
        for b in range(1, NB):
            acck = acck + soh[b] * w1k_ref[b, c]
            accv = accv + soh[b] * w1v_ref[b, c]
        hk = acck * (1.0 / (1.0 + jnp.exp(-acck)))
        hvs.append(accv * (1.0 / (1.0 + jnp.exp(-accv))))
        dotqk = dotqk + hk * p2[:, c * _N:(c + 1) * _N]

    ex = jnp.where(mask, cutoff * jnp.exp(dotqk), 0.0)
    z = jnp.sum(ex, axis=1, keepdims=True)
    z = jnp.where(z == 0.0, 1.0, z)
    w = jnp.sqrt(ex / z)

    g = jnp.concatenate([w * hv for hv in hvs], axis=1)

    o0_ref[0] = jnp.dot(g, vtab[:, :14], precision=MED)
    for s, u in enumerate((ux, uy, uz)):
        ut = jnp.concatenate([u] * 16, axis=1)
        o1_ref[0, s] = jnp.dot(g * ut, vtab[:, 14:20], precision=MED)


def _mha_body(y_ref, wq_ref, wk_ref, wv_ref, bq_ref, bk_ref, bv_ref,
              wot_ref, bo_ref, o_ref):
    h = pl.program_id(1)
    yb = y_ref[0]
    cdims = (((1,), (1,)), ((), ()))
    q = jax.lax.dot_general(yb, wq_ref[0], cdims, precision=HI) + bq_ref[0]
    k = jax.lax.dot_general(yb, wk_ref[0], cdims, precision=HI) + bk_ref[0]
    v = jax.lax.dot_general(yb, wv_ref[0], cdims, precision=HI) + bv_ref[0]
    s = jax.lax.dot_general(q, k, cdims, precision=MED) * 0.25
    s = s - jnp.max(s, axis=1, keepdims=True)
    p = jnp.exp(s)
    p = p / jnp.sum(p, axis=1, keepdims=True)
    oh = jnp.dot(p, v, precision=MED)
    contrib = jnp.dot(oh, wot_ref[0], precision=MED)

    @pl.when(h == 0)
    def _():
        o_ref[0] = contrib + bo_ref[...]

    @pl.when(h != 0)
    def _():
        o_ref[0] = o_ref[0] + contrib


def kernel(x, y, W_hq, W1k, W2k, W1v, W2v, W_dot, in_proj_w, in_proj_b,
           out_proj_w, out_proj_b):
    B, N, D = x.shape
    Fi = D - 3
    nrm = math.sqrt(Fi)
    s_q = 1.0 / (4.0 * Fi * math.sqrt(D * 12.0))
    s_v = 1.0 / (4.0 * nrm)

    clouds = jnp.concatenate([x, y], axis=0)
    NC = 2 * B
    pos = clouds[:, :, :3]
    f = clouds[:, :, 3:]
    fpad = jnp.pad(f, ((0, 0), (0, 0), (0, D - Fi)))
    post = jnp.swapaxes(pos, 1, 2)

    wkr = W2k[:, :Fi * 12].reshape(16, Fi, 12).transpose(2, 0, 1)
    wkrt = jnp.pad(wkr.reshape(192, Fi), ((0, 0), (0, D - Fi)))
    v0 = W2v[:, :Fi * 14].reshape(16, Fi, 14) * s_v
    v1 = W2v[:, Fi * 14:].reshape(16, Fi, 6) * (s_v * math.sqrt(3.0))
    wv_all = jnp.pad(jnp.concatenate([v0, v1], axis=2),
                     ((0, 0), (0, D - Fi), (0, 0)))
    whq_pad = jnp.pad(W_hq, ((0, D - Fi), (0, 0)))

    akt, vtab, qws = pl.pallas_call(
        functools.partial(_prep_body, s_q=s_q),
        grid=(NC,),
        in_specs=[
            pl.BlockSpec((1, N, D), lambda b: (b, 0, 0)),
            pl.BlockSpec((192, D), lambda b: (0, 0)),
            pl.BlockSpec((16, D, 20), lambda b: (0, 0, 0)),
            pl.BlockSpec((D, D), lambda b: (0, 0)),
            pl.BlockSpec((D, 12), lambda b: (0, 0)),
        ],
        out_specs=[
            pl.BlockSpec((1, 192, N), lambda b: (b, 0, 0)),
            pl.BlockSpec((1, 16, N, 20), lambda b: (b, 0, 0, 0)),
            pl.BlockSpec((1, N, 12), lambda b: (b, 0, 0)),
        ],
        out_shape=[
            jax.ShapeDtypeStruct((NC, 192, N), jnp.float32),
            jax.ShapeDtypeStruct((NC, 16, N, 20), jnp.float32),
            jax.ShapeDtypeStruct((NC, N, 12), jnp.float32),
        ],
    )(fpad, wkrt, wv_all, whq_pad, W_dot)

    akt = akt.reshape(NC, 12, 16 * N)
    vtab = vtab.reshape(NC, 16 * N, 20)

    nchunk = N // _C
    o0, o1 = pl.pallas_call(
        _edge_body,
        grid=(NC, nchunk),
        in_specs=[
            pl.BlockSpec((1, 3, N), lambda b, j: (b, 0, 0)),
            pl.BlockSpec((1, _C, 3), lambda b, j: (b, j, 0)),
            pl.BlockSpec((1, _C, 12), lambda b, j: (b, j, 0)),
            pl.BlockSpec((1, 12, 16 * N), lambda b, j: (b, 0, 0)),
            pl.BlockSpec((1, 16 * N, 20), lambda b, j: (b, 0, 0)),
            pl.BlockSpec(memory_space=pltpu.SMEM),
            pl.BlockSpec(memory_space=pltpu.SMEM),
        ],
        out_specs=[
            pl.BlockSpec((1, _C, 14), lambda b, j: (b, j, 0)),
            pl.BlockSpec((1, 3, _C, 6), lambda b, j: (b, 0, j, 0)),
        ],
        out_shape=[
            jax.ShapeDtypeStruct((NC, N, 14), jnp.float32),
            jax.ShapeDtypeStruct((NC, 3, N, 6), jnp.float32),
        ],
    )(post, pos, qws, akt, vtab, W1k, W1v)

    o1r = jnp.swapaxes(o1, 1, 2).reshape(NC, N, 3, 6)
    o1r = jnp.swapaxes(o1r, 2, 3).reshape(NC, N, 18)
    esa_out = jnp.concatenate([o0, o1r], axis=-1)
    xi = esa_out[:B]
    yi = esa_out[B:]

    dh = D // NHEAD
    wq = in_proj_w[:D].reshape(NHEAD, dh, D)
    wk = in_proj_w[D:2 * D].reshape(NHEAD, dh, D)
    wv = in_proj_w[2 * D:].reshape(NHEAD, dh, D)
    bq = in_proj_b[:D].reshape(NHEAD, 1, dh)
    bk = in_proj_b[D:2 * D].reshape(NHEAD, 1, dh)
    bv = in_proj_b[2 * D:].reshape(NHEAD, 1, dh)
    wot = jnp.swapaxes(out_proj_w, 0, 1).reshape(NHEAD, dh, D)
    bo = out_proj_b.reshape(1, D)

    yii = pl.pallas_call(
        _mha_body,
        grid=(B, NHEAD),
        in_specs=[
            pl.BlockSpec((1, N, D), lambda b, h: (b, 0, 0)),
            pl.BlockSpec((1, dh, D), lambda b, h: (h, 0, 0)),
            pl.BlockSpec((1, dh, D), lambda b, h: (h, 0, 0)),
            pl.BlockSpec((1, dh, D), lambda b, h: (h, 0, 0)),
            pl.BlockSpec((1, 1, dh), lambda b, h: (h, 0, 0)),
            pl.BlockSpec((1, 1, dh), lambda b, h: (h, 0, 0)),
            pl.BlockSpec((1, 1, dh), lambda b, h: (h, 0, 0)),
            pl.BlockSpec((1, dh, D), lambda b, h: (h, 0, 0)),
            pl.BlockSpec((1, D), lambda b, h: (0, 0)),
        ],
        out_specs=pl.BlockSpec((1, N, D), lambda b, h: (b, 0, 0)),
        out_shape=jax.ShapeDtypeStruct((B, N, D), jnp.float32),
    )(y, wq, wk, wv, bq, bk, bv, wot, bo)

    return jnp.concatenate([xi, yi, yii], axis=-1)
